# Initial kernel scaffold; baseline (speedup 1.0000x reference)
import jax
import jax.numpy as jnp
from jax.experimental import pallas as pl


def kernel(x, edge_index, perm, dist_w, W1, b1, W2, b2):
    raise NotImplementedError("write your pallas kernel here")


# speedup vs baseline: 10.1072x; 10.1072x over previous
import math

import jax
import jax.numpy as jnp
import numpy as np
from jax.experimental import pallas as pl

N = 10000
E0 = 320000
OUT = 16
K_TAYLOR = 10


def _scale_kernel(acc_ref, o_ref):
    o_ref[...] = acc_ref[...] * np.float32(math.exp(-1.0))


def kernel(x, edge_index, perm, dist_w, W1, b1, W2, b2):
    h = jnp.maximum(x @ W1.T + b1, 0.0) @ W2.T + b2
    y0 = h.T

    src = edge_index[0]
    dst = edge_index[1]
    dF = jax.nn.sigmoid((dist_w[perm] + dist_w) * 0.5)
    dFv = jnp.concatenate([dF[:, 0], jnp.ones((N,), dF.dtype)])
    deg = jax.ops.segment_sum(dFv, dst, num_segments=N)
    a = dFv * jax.lax.rsqrt(deg[dst] * deg[src])

    def apply_P(y):
        return jax.ops.segment_sum((a[None, :] * y[:, src]).T, dst,
                                   num_segments=N).T

    v = y0
    acc = y0
    for k in range(1, K_TAYLOR + 1):
        v = apply_P(v)
        acc = acc + v * np.float32(1.0 / math.factorial(k))

    return pl.pallas_call(
        _scale_kernel,
        out_shape=jax.ShapeDtypeStruct((OUT, N), jnp.float32),
    )(acc)


# speedup vs baseline: 343.9269x; 34.0279x over previous
import functools
import math

import jax
import jax.numpy as jnp
import numpy as np
from jax import lax
from jax.experimental import pallas as pl
from jax.experimental.pallas import tpu as pltpu
from jax.experimental.pallas import tpu_sc as plsc

N = 10000
E0 = 320000
INP = 128
OUT = 16
K_TAYLOR = 10

NC, NS = 2, 16
N_PAD = 10240
ROWS_PER_TILE = N_PAD // NS
E = E0 + N
E_PAD = 330240
EPT_B = E_PAD // (NC * NS)
CH_B = 2064
EPT_A0 = E0 // NS
CH_A0 = 2000
EPT_A1 = E_PAD // NS
CH_A1 = 2064


def _mlp_body(x_ref, w1t_ref, b1_ref, w2t_ref, b2_ref, o_ref):
    h = jnp.dot(x_ref[...], w1t_ref[...], preferred_element_type=jnp.float32)
    h = jnp.maximum(h + b1_ref[...], 0.0)
    o = jnp.dot(h, w2t_ref[...], preferred_element_type=jnp.float32)
    o_ref[...] = o + b2_ref[...]


def _mlp(x_pad, W1, b1, W2, b2):
    return pl.pallas_call(
        _mlp_body,
        out_shape=jax.ShapeDtypeStruct((N_PAD, OUT), jnp.float32),
    )(x_pad, W1.T, b1.reshape(1, -1), W2.T, b2.reshape(1, -1))


def _rsqrt_newton(x):
    i = lax.bitcast_convert_type(x, jnp.int32)
    i = jnp.int32(0x5F3759DF) - lax.shift_right_arithmetic(i, 1)
    y = lax.bitcast_convert_type(i, jnp.float32)
    for _ in range(4):
        y = y * (1.5 - 0.5 * x * y * y)
    return y


_SC_MESH = plsc.VectorSubcoreMesh(core_axis_name="c", subcore_axis_name="s")


@functools.partial(
    pl.kernel,
    out_type=jax.ShapeDtypeStruct((E_PAD,), jnp.float32),
    mesh=_SC_MESH,
    compiler_params=pltpu.CompilerParams(use_tc_tiling_on_sc=False),
    scratch_types=[
        pltpu.VMEM_SHARED((E_PAD,), jnp.float32),
        pltpu.VMEM_SHARED((N_PAD,), jnp.float32),
        pltpu.VMEM((CH_A0,), jnp.int32),
        pltpu.VMEM((CH_A0,), jnp.int32),
        pltpu.VMEM((CH_A0,), jnp.float32),
        pltpu.VMEM((CH_A0,), jnp.float32),
        pltpu.VMEM((CH_A1,), jnp.int32),
        pltpu.VMEM((CH_A1,), jnp.int32),
        pltpu.VMEM((CH_A1,), jnp.float32),
        pltpu.VMEM((CH_A1,), jnp.float32),
        pltpu.VMEM((CH_A1,), jnp.float32),
        pltpu.VMEM((ROWS_PER_TILE,), jnp.float32),
        pltpu.SemaphoreType.DMA,
    ],
)
def _prep_kernel(perm_hbm, dw_hbm, src_hbm, dst_hbm, a_hbm,
                 a_s, deg_s, idx1, idxd1, d1a, d1b,
                 idx3a, idx3b, d3a, d3b, d3c, ndeg, sem):
    cid = lax.axis_index("c")
    tid = lax.axis_index("s")
    lanes = lax.iota(jnp.int32, 16)

    @pl.when(cid == 0)
    def _():
        nbase = tid * ROWS_PER_TILE
        for j in range(ROWS_PER_TILE // 16):
            ndeg[pl.ds(16 * j, 16)] = jnp.full((16,), 1.0, jnp.float32)
        pltpu.sync_copy(ndeg, deg_s.at[pl.ds(nbase, ROWS_PER_TILE)])
        for j in range(ROWS_PER_TILE // 16):
            gidx = nbase + 16 * j + lanes
            ndeg[pl.ds(16 * j, 16)] = jnp.where(
                gidx < N, 1.0, 0.0).astype(jnp.float32)
        pltpu.sync_copy(ndeg, a_s.at[pl.ds(E0 + nbase, ROWS_PER_TILE)])
        plsc.subcore_barrier()

        def phase1(i, _):
            ebase = tid * EPT_A0 + i * CH_A0
            pltpu.sync_copy(dw_hbm.at[pl.ds(ebase, CH_A0)], d1a)
            pltpu.sync_copy(perm_hbm.at[pl.ds(ebase, CH_A0)], idx1)
            pltpu.async_copy(dw_hbm.at[idx1], d1b, sem).wait()

            def sig(j, _):
                u = (d1a[pl.ds(16 * j, 16)] + d1b[pl.ds(16 * j, 16)]) * 0.5
                s = 1.0 / (1.0 + jnp.exp(-u))
                d1a[pl.ds(16 * j, 16)] = s
                return 0
            lax.fori_loop(0, CH_A0 // 16, sig, 0, unroll=4)
            pltpu.sync_copy(d1a, a_s.at[pl.ds(ebase, CH_A0)])
            pltpu.sync_copy(dst_hbm.at[pl.ds(ebase, CH_A0)], idxd1)
            pltpu.sync_copy(d1a, deg_s.at[idxd1], add=True)
            return 0
        lax.fori_loop(0, EPT_A0 // CH_A0, phase1, 0)
        plsc.subcore_barrier()

        nbase = tid * ROWS_PER_TILE
        pltpu.sync_copy(deg_s.at[pl.ds(nbase, ROWS_PER_TILE)], ndeg)

        def rsq(j, _):
            ndeg[pl.ds(16 * j, 16)] = _rsqrt_newton(ndeg[pl.ds(16 * j, 16)])
            return 0
        lax.fori_loop(0, ROWS_PER_TILE // 16, rsq, 0, unroll=4)
        pltpu.sync_copy(ndeg, deg_s.at[pl.ds(nbase, ROWS_PER_TILE)])
        plsc.subcore_barrier()

        def phase3(i, _):
            ebase = tid * EPT_A1 + i * CH_A1
            pltpu.sync_copy(src_hbm.at[pl.ds(ebase, CH_A1)], idx3a)
            pltpu.sync_copy(dst_hbm.at[pl.ds(ebase, CH_A1)], idx3b)
            pltpu.async_copy(deg_s.at[idx3a], d3b, sem).wait()
            pltpu.async_copy(deg_s.at[idx3b], d3c, sem).wait()
            pltpu.sync_copy(a_s.at[pl.ds(ebase, CH_A1)], d3a)

            def fin(j, _):
                sl = pl.ds(16 * j, 16)
                d3a[sl] = d3a[sl] * (d3b[sl] * d3c[sl])
                return 0
            lax.fori_loop(0, CH_A1 // 16, fin, 0, unroll=4)
            pltpu.sync_copy(d3a, a_hbm.at[pl.ds(ebase, CH_A1)])
            return 0
        lax.fori_loop(0, EPT_A1 // CH_A1, phase3, 0)


@functools.partial(
    pl.kernel,
    out_type=jax.ShapeDtypeStruct((NC, N_PAD, OUT), jnp.float32),
    mesh=_SC_MESH,
    compiler_params=pltpu.CompilerParams(use_tc_tiling_on_sc=False),
    scratch_types=[
        pltpu.VMEM_SHARED((N_PAD, OUT), jnp.float32),
        pltpu.VMEM_SHARED((N_PAD, OUT), jnp.float32),
        pltpu.VMEM((CH_B,), jnp.int32),
        pltpu.VMEM((CH_B,), jnp.int32),
        pltpu.VMEM((CH_B,), jnp.float32),
        pltpu.VMEM((CH_B, OUT), jnp.float32),
        pltpu.VMEM((ROWS_PER_TILE, OUT), jnp.float32),
        pltpu.SemaphoreType.DMA,
    ],
)
def _spmm_kernel(v_hbm, src_hbm, dst_hbm, a_hbm, out_hbm,
                 v_s, nv_s, idx_a, idx_b, dat_a, rows, zrows, sem):
    cid = lax.axis_index("c")
    tid = lax.axis_index("s")
    nbase = tid * ROWS_PER_TILE

    pltpu.sync_copy(v_hbm.at[pl.ds(nbase, ROWS_PER_TILE)],
                    v_s.at[pl.ds(nbase, ROWS_PER_TILE)])

    def z(j, _):
        zrows[j, :] = jnp.zeros((16,), jnp.float32)
        return 0
    lax.fori_loop(0, ROWS_PER_TILE, z, 0, unroll=8)
    pltpu.sync_copy(zrows, nv_s.at[pl.ds(nbase, ROWS_PER_TILE)])
    plsc.subcore_barrier()

    def chunk(i, _):
        ebase = (cid * NS + tid) * EPT_B + i * CH_B
        pltpu.sync_copy(src_hbm.at[pl.ds(ebase, CH_B)], idx_a)
        pltpu.sync_copy(dst_hbm.at[pl.ds(ebase, CH_B)], idx_b)
        pltpu.sync_copy(a_hbm.at[pl.ds(ebase, CH_B)], dat_a)
        pltpu.async_copy(v_s.at[idx_a], rows, sem).wait()

        def scale(g, _):
            coefs = dat_a[pl.ds(16 * g, 16)]
            for lane in range(16):
                j = 16 * g + lane
                rows[j, :] = rows[j, :] * coefs[lane]
            return 0
        lax.fori_loop(0, CH_B // 16, scale, 0)
        pltpu.sync_copy(rows, nv_s.at[idx_b], add=True)
        return 0
    lax.fori_loop(0, EPT_B // CH_B, chunk, 0)
    plsc.subcore_barrier()

    pltpu.sync_copy(nv_s.at[pl.ds(nbase, ROWS_PER_TILE)],
                    out_hbm.at[cid, pl.ds(nbase, ROWS_PER_TILE)])


def kernel(x, edge_index, perm, dist_w, W1, b1, W2, b2):
    x_pad = jnp.concatenate(
        [x, jnp.zeros((N_PAD - N, INP), jnp.float32)], axis=0)
    y0 = _mlp(x_pad, W1, b1, W2, b2)

    pad_e = jnp.zeros((E_PAD - E,), jnp.int32)
    src_p = jnp.concatenate([edge_index[0].astype(jnp.int32), pad_e])
    dst_p = jnp.concatenate([edge_index[1].astype(jnp.int32), pad_e])
    dw = dist_w[:, 0]
    a_e = _prep_kernel(perm.astype(jnp.int32), dw, src_p, dst_p)

    inv_e = np.float32(math.exp(-1.0))
    v = y0
    acc = y0 * inv_e
    for k in range(1, K_TAYLOR + 1):
        parts = _spmm_kernel(v, src_p, dst_p, a_e)
        v = parts[0] + parts[1]
        acc = acc + v * np.float32(math.exp(-1.0) / math.factorial(k))

    return acc[:N].T


# speedup vs baseline: 403.2147x; 1.1724x over previous
import functools
import math

import jax
import jax.numpy as jnp
import numpy as np
from jax import lax
from jax.experimental import pallas as pl
from jax.experimental.pallas import tpu as pltpu
from jax.experimental.pallas import tpu_sc as plsc

N = 10000
E0 = 320000
INP = 128
OUT = 16
K_TAYLOR = 8

NC, NS = 2, 16
N_PAD = 10240
ROWS_PER_TILE = N_PAD // NS
E = E0 + N
E_PAD = 330240
EPT_B = E_PAD // (NC * NS)
CH_B = 2064
EPT_A0 = E0 // NS
CH_A0 = 2000
EPT_A1 = E_PAD // NS
CH_A1 = 2064


def _mlp_body(x_ref, w1t_ref, b1_ref, w2t_ref, b2_ref, o_ref):
    h = jnp.dot(x_ref[...], w1t_ref[...], preferred_element_type=jnp.float32)
    h = jnp.maximum(h + b1_ref[...], 0.0)
    o = jnp.dot(h, w2t_ref[...], preferred_element_type=jnp.float32)
    o_ref[...] = o + b2_ref[...]


def _mlp(x_pad, W1, b1, W2, b2):
    return pl.pallas_call(
        _mlp_body,
        out_shape=jax.ShapeDtypeStruct((N_PAD, OUT), jnp.float32),
    )(x_pad, W1.T, b1.reshape(1, -1), W2.T, b2.reshape(1, -1))


def _rsqrt_newton(x):
    i = lax.bitcast_convert_type(x, jnp.int32)
    i = jnp.int32(0x5F3759DF) - lax.shift_right_arithmetic(i, 1)
    y = lax.bitcast_convert_type(i, jnp.float32)
    for _ in range(4):
        y = y * (1.5 - 0.5 * x * y * y)
    return y


_SC_MESH = plsc.VectorSubcoreMesh(core_axis_name="c", subcore_axis_name="s")


@functools.partial(
    pl.kernel,
    out_type=jax.ShapeDtypeStruct((E_PAD,), jnp.float32),
    mesh=_SC_MESH,
    compiler_params=pltpu.CompilerParams(use_tc_tiling_on_sc=False),
    scratch_types=[
        pltpu.VMEM_SHARED((E_PAD,), jnp.float32),
        pltpu.VMEM_SHARED((N_PAD,), jnp.float32),
        pltpu.VMEM((CH_A0,), jnp.int32),
        pltpu.VMEM((CH_A0,), jnp.int32),
        pltpu.VMEM((CH_A0,), jnp.float32),
        pltpu.VMEM((CH_A0,), jnp.float32),
        pltpu.VMEM((CH_A1,), jnp.int32),
        pltpu.VMEM((CH_A1,), jnp.int32),
        pltpu.VMEM((CH_A1,), jnp.float32),
        pltpu.VMEM((CH_A1,), jnp.float32),
        pltpu.VMEM((CH_A1,), jnp.float32),
        pltpu.VMEM((ROWS_PER_TILE,), jnp.float32),
        pltpu.SemaphoreType.DMA,
    ],
)
def _prep_kernel(perm_hbm, dw_hbm, src_hbm, dst_hbm, a_hbm,
                 a_s, deg_s, idx1, idxd1, d1a, d1b,
                 idx3a, idx3b, d3a, d3b, d3c, ndeg, sem):
    cid = lax.axis_index("c")
    tid = lax.axis_index("s")
    lanes = lax.iota(jnp.int32, 16)

    @pl.when(cid == 0)
    def _():
        nbase = tid * ROWS_PER_TILE
        for j in range(ROWS_PER_TILE // 16):
            ndeg[pl.ds(16 * j, 16)] = jnp.full((16,), 1.0, jnp.float32)
        pltpu.sync_copy(ndeg, deg_s.at[pl.ds(nbase, ROWS_PER_TILE)])
        for j in range(ROWS_PER_TILE // 16):
            gidx = nbase + 16 * j + lanes
            ndeg[pl.ds(16 * j, 16)] = jnp.where(
                gidx < N, 1.0, 0.0).astype(jnp.float32)
        pltpu.sync_copy(ndeg, a_s.at[pl.ds(E0 + nbase, ROWS_PER_TILE)])
        plsc.subcore_barrier()

        def phase1(i, _):
            ebase = tid * EPT_A0 + i * CH_A0
            pltpu.sync_copy(dw_hbm.at[pl.ds(ebase, CH_A0)], d1a)
            pltpu.sync_copy(perm_hbm.at[pl.ds(ebase, CH_A0)], idx1)
            pltpu.async_copy(dw_hbm.at[idx1], d1b, sem).wait()

            def sig(j, _):
                u = (d1a[pl.ds(16 * j, 16)] + d1b[pl.ds(16 * j, 16)]) * 0.5
                s = 1.0 / (1.0 + jnp.exp(-u))
                d1a[pl.ds(16 * j, 16)] = s
                return 0
            lax.fori_loop(0, CH_A0 // 16, sig, 0, unroll=4)
            pltpu.sync_copy(d1a, a_s.at[pl.ds(ebase, CH_A0)])
            pltpu.sync_copy(dst_hbm.at[pl.ds(ebase, CH_A0)], idxd1)
            pltpu.sync_copy(d1a, deg_s.at[idxd1], add=True)
            return 0
        lax.fori_loop(0, EPT_A0 // CH_A0, phase1, 0)
        plsc.subcore_barrier()

        nbase = tid * ROWS_PER_TILE
        pltpu.sync_copy(deg_s.at[pl.ds(nbase, ROWS_PER_TILE)], ndeg)

        def rsq(j, _):
            ndeg[pl.ds(16 * j, 16)] = _rsqrt_newton(ndeg[pl.ds(16 * j, 16)])
            return 0
        lax.fori_loop(0, ROWS_PER_TILE // 16, rsq, 0, unroll=4)
        pltpu.sync_copy(ndeg, deg_s.at[pl.ds(nbase, ROWS_PER_TILE)])
        plsc.subcore_barrier()

        def phase3(i, _):
            ebase = tid * EPT_A1 + i * CH_A1
            pltpu.sync_copy(src_hbm.at[pl.ds(ebase, CH_A1)], idx3a)
            pltpu.sync_copy(dst_hbm.at[pl.ds(ebase, CH_A1)], idx3b)
            pltpu.async_copy(deg_s.at[idx3a], d3b, sem).wait()
            pltpu.async_copy(deg_s.at[idx3b], d3c, sem).wait()
            pltpu.sync_copy(a_s.at[pl.ds(ebase, CH_A1)], d3a)

            def fin(j, _):
                sl = pl.ds(16 * j, 16)
                d3a[sl] = d3a[sl] * (d3b[sl] * d3c[sl])
                return 0
            lax.fori_loop(0, CH_A1 // 16, fin, 0, unroll=4)
            pltpu.sync_copy(d3a, a_hbm.at[pl.ds(ebase, CH_A1)])
            return 0
        lax.fori_loop(0, EPT_A1 // CH_A1, phase3, 0)


@functools.partial(
    pl.kernel,
    out_type=(
        jax.ShapeDtypeStruct((NC, N_PAD, OUT), jnp.float32),
        jax.ShapeDtypeStruct((N_PAD, OUT), jnp.float32),
    ),
    mesh=_SC_MESH,
    compiler_params=pltpu.CompilerParams(use_tc_tiling_on_sc=False),
    scratch_types=[
        pltpu.VMEM_SHARED((N_PAD, OUT), jnp.float32),
        pltpu.VMEM_SHARED((N_PAD, OUT), jnp.float32),
        pltpu.VMEM((CH_B,), jnp.int32),
        pltpu.VMEM((CH_B,), jnp.int32),
        pltpu.VMEM((CH_B,), jnp.float32),
        pltpu.VMEM((CH_B, OUT), jnp.float32),
        pltpu.VMEM((ROWS_PER_TILE, OUT), jnp.float32),
        pltpu.VMEM((ROWS_PER_TILE, OUT), jnp.float32),
        pltpu.VMEM((16,), jnp.float32),
        pltpu.SemaphoreType.DMA,
    ],
)
def _spmm_kernel(parts_hbm, acc_hbm, ck_hbm, src_hbm, dst_hbm, a_hbm,
                 out_hbm, accout_hbm,
                 v_s, nv_s, idx_a, idx_b, dat_a, rows, p0buf, p1buf, ckbuf,
                 sem):
    cid = lax.axis_index("c")
    tid = lax.axis_index("s")
    nbase = tid * ROWS_PER_TILE

    pltpu.sync_copy(parts_hbm.at[0, pl.ds(nbase, ROWS_PER_TILE)], p0buf)
    pltpu.sync_copy(parts_hbm.at[1, pl.ds(nbase, ROWS_PER_TILE)], p1buf)
    pltpu.sync_copy(ck_hbm, ckbuf)

    def comb(j, _):
        p0buf[j, :] = p0buf[j, :] + p1buf[j, :]
        return 0
    lax.fori_loop(0, ROWS_PER_TILE, comb, 0, unroll=8)
    pltpu.sync_copy(p0buf, v_s.at[pl.ds(nbase, ROWS_PER_TILE)])

    @pl.when(cid == 0)
    def _():
        pltpu.sync_copy(acc_hbm.at[pl.ds(nbase, ROWS_PER_TILE)], p1buf)
        ck = ckbuf[pl.ds(0, 16)]

        def accu(j, _):
            p1buf[j, :] = p1buf[j, :] + ck * p0buf[j, :]
            return 0
        lax.fori_loop(0, ROWS_PER_TILE, accu, 0, unroll=8)
        pltpu.sync_copy(p1buf, accout_hbm.at[pl.ds(nbase, ROWS_PER_TILE)])

    def z(j, _):
        p1buf[j, :] = jnp.zeros((16,), jnp.float32)
        return 0
    lax.fori_loop(0, ROWS_PER_TILE, z, 0, unroll=8)
    pltpu.sync_copy(p1buf, nv_s.at[pl.ds(nbase, ROWS_PER_TILE)])
    plsc.subcore_barrier()

    def chunk(i, _):
        ebase = (cid * NS + tid) * EPT_B + i * CH_B
        pltpu.sync_copy(src_hbm.at[pl.ds(ebase, CH_B)], idx_a)
        pltpu.sync_copy(dst_hbm.at[pl.ds(ebase, CH_B)], idx_b)
        pltpu.sync_copy(a_hbm.at[pl.ds(ebase, CH_B)], dat_a)
        pltpu.async_copy(v_s.at[idx_a], rows, sem).wait()

        def scale(g, _):
            coefs = dat_a[pl.ds(16 * g, 16)]
            for lane in range(16):
                j = 16 * g + lane
                rows[j, :] = rows[j, :] * coefs[lane]
            return 0
        lax.fori_loop(0, CH_B // 16, scale, 0)
        pltpu.sync_copy(rows, nv_s.at[idx_b], add=True)
        return 0
    lax.fori_loop(0, EPT_B // CH_B, chunk, 0)
    plsc.subcore_barrier()

    pltpu.sync_copy(nv_s.at[pl.ds(nbase, ROWS_PER_TILE)],
                    out_hbm.at[cid, pl.ds(nbase, ROWS_PER_TILE)])


def kernel(x, edge_index, perm, dist_w, W1, b1, W2, b2):
    x_pad = jnp.concatenate(
        [x, jnp.zeros((N_PAD - N, INP), jnp.float32)], axis=0)
    y0 = _mlp(x_pad, W1, b1, W2, b2)

    pad_e = jnp.zeros((E_PAD - E,), jnp.int32)
    src_p = jnp.concatenate([edge_index[0].astype(jnp.int32), pad_e])
    dst_p = jnp.concatenate([edge_index[1].astype(jnp.int32), pad_e])
    dw = dist_w[:, 0]
    a_e = _prep_kernel(perm.astype(jnp.int32), dw, src_p, dst_p)

    parts = jnp.stack([y0, jnp.zeros((N_PAD, OUT), jnp.float32)])
    acc = jnp.zeros((N_PAD, OUT), jnp.float32)
    for j in range(K_TAYLOR + 1):
        ckvec = jnp.asarray(
            np.full((16,), math.exp(-1.0) / math.factorial(j), np.float32))
        parts, acc = _spmm_kernel(parts, acc, ckvec, src_p, dst_p, a_e)

    return acc[:N].T


# speedup vs baseline: 533.8903x; 1.3241x over previous
import functools
import math

import jax
import jax.numpy as jnp
import numpy as np
from jax import lax
from jax.experimental import pallas as pl
from jax.experimental.pallas import tpu as pltpu
from jax.experimental.pallas import tpu_sc as plsc

N = 10000
E0 = 320000
INP = 128
OUT = 16
K_TAYLOR = 8

NC, NS = 2, 16
N_PAD = 10240
ROWS_PER_TILE = N_PAD // NS
E = E0 + N
E_PAD = 330240
EPT_B = E_PAD // (NC * NS)
CH_B = 2064
EPT_A0 = E0 // NS
CH_A0 = 2000
EPT_A1 = E_PAD // NS
CH_A1 = 2064


def _mlp_body(x_ref, w1t_ref, b1_ref, w2t_ref, b2_ref, o_ref):
    h = jnp.dot(x_ref[...], w1t_ref[...], preferred_element_type=jnp.float32)
    h = jnp.maximum(h + b1_ref[...], 0.0)
    o = jnp.dot(h, w2t_ref[...], preferred_element_type=jnp.float32)
    o_ref[0, :, :] = o + b2_ref[...]
    o_ref[1, :, :] = jnp.zeros((N_PAD, OUT), jnp.float32)


def _mlp(x_pad, W1, b1, W2, b2):
    return pl.pallas_call(
        _mlp_body,
        out_shape=jax.ShapeDtypeStruct((NC, N_PAD, OUT), jnp.float32),
    )(x_pad, W1.T, b1.reshape(1, -1), W2.T, b2.reshape(1, -1))


def _rsqrt_newton(x):
    i = lax.bitcast_convert_type(x, jnp.int32)
    i = jnp.int32(0x5F3759DF) - lax.shift_right_arithmetic(i, 1)
    y = lax.bitcast_convert_type(i, jnp.float32)
    for _ in range(4):
        y = y * (1.5 - 0.5 * x * y * y)
    return y


_SC_MESH = plsc.VectorSubcoreMesh(core_axis_name="c", subcore_axis_name="s")


@functools.partial(
    pl.kernel,
    out_type=jax.ShapeDtypeStruct((E_PAD,), jnp.float32),
    mesh=_SC_MESH,
    compiler_params=pltpu.CompilerParams(use_tc_tiling_on_sc=False),
    scratch_types=[
        pltpu.VMEM_SHARED((E_PAD,), jnp.float32),
        pltpu.VMEM_SHARED((E0,), jnp.float32),
        pltpu.VMEM_SHARED((N_PAD,), jnp.float32),
        pltpu.VMEM((CH_A0,), jnp.int32),
        pltpu.VMEM((CH_A0,), jnp.int32),
        pltpu.VMEM((CH_A0,), jnp.float32),
        pltpu.VMEM((CH_A0,), jnp.float32),
        pltpu.VMEM((CH_A1,), jnp.int32),
        pltpu.VMEM((CH_A1,), jnp.int32),
        pltpu.VMEM((CH_A1,), jnp.float32),
        pltpu.VMEM((CH_A1,), jnp.float32),
        pltpu.VMEM((CH_A1,), jnp.float32),
        pltpu.VMEM((ROWS_PER_TILE,), jnp.float32),
        pltpu.SemaphoreType.DMA,
    ],
)
def _prep_kernel(perm_hbm, dw_hbm, src_hbm, dst_hbm, a_hbm,
                 a_s, dw_s, deg_s, idx1, idxd1, d1a, d1b,
                 idx3a, idx3b, d3a, d3b, d3c, ndeg, sem):
    cid = lax.axis_index("c")
    tid = lax.axis_index("s")
    lanes = lax.iota(jnp.int32, 16)

    @pl.when(cid == 0)
    def _():
        nbase = tid * ROWS_PER_TILE
        pltpu.sync_copy(dw_hbm.at[pl.ds(tid * EPT_A0, EPT_A0)],
                        dw_s.at[pl.ds(tid * EPT_A0, EPT_A0)])
        for j in range(ROWS_PER_TILE // 16):
            ndeg[pl.ds(16 * j, 16)] = jnp.full((16,), 1.0, jnp.float32)
        pltpu.sync_copy(ndeg, deg_s.at[pl.ds(nbase, ROWS_PER_TILE)])
        for j in range(ROWS_PER_TILE // 16):
            gidx = nbase + 16 * j + lanes
            ndeg[pl.ds(16 * j, 16)] = jnp.where(
                gidx < N, 1.0, 0.0).astype(jnp.float32)
        pltpu.sync_copy(ndeg, a_s.at[pl.ds(E0 + nbase, ROWS_PER_TILE)])
        plsc.subcore_barrier()

        def phase1(i, _):
            ebase = tid * EPT_A0 + i * CH_A0
            pltpu.sync_copy(dw_hbm.at[pl.ds(ebase, CH_A0)], d1a)
            pltpu.sync_copy(perm_hbm.at[pl.ds(ebase, CH_A0)], idx1)
            pltpu.async_copy(dw_s.at[idx1], d1b, sem).wait()

            def sig(j, _):
                u = (d1a[pl.ds(16 * j, 16)] + d1b[pl.ds(16 * j, 16)]) * 0.5
                s = 1.0 / (1.0 + jnp.exp(-u))
                d1a[pl.ds(16 * j, 16)] = s
                return 0
            lax.fori_loop(0, CH_A0 // 16, sig, 0, unroll=4)
            pltpu.sync_copy(d1a, a_s.at[pl.ds(ebase, CH_A0)])
            pltpu.sync_copy(dst_hbm.at[pl.ds(ebase, CH_A0)], idxd1)
            pltpu.sync_copy(d1a, deg_s.at[idxd1], add=True)
            return 0
        lax.fori_loop(0, EPT_A0 // CH_A0, phase1, 0)
        plsc.subcore_barrier()

        nbase = tid * ROWS_PER_TILE
        pltpu.sync_copy(deg_s.at[pl.ds(nbase, ROWS_PER_TILE)], ndeg)

        def rsq(j, _):
            ndeg[pl.ds(16 * j, 16)] = _rsqrt_newton(ndeg[pl.ds(16 * j, 16)])
            return 0
        lax.fori_loop(0, ROWS_PER_TILE // 16, rsq, 0, unroll=4)
        pltpu.sync_copy(ndeg, deg_s.at[pl.ds(nbase, ROWS_PER_TILE)])
        plsc.subcore_barrier()

        def phase3(i, _):
            ebase = tid * EPT_A1 + i * CH_A1
            pltpu.sync_copy(src_hbm.at[pl.ds(ebase, CH_A1)], idx3a)
            pltpu.sync_copy(dst_hbm.at[pl.ds(ebase, CH_A1)], idx3b)
            pltpu.async_copy(deg_s.at[idx3a], d3b, sem).wait()
            pltpu.async_copy(deg_s.at[idx3b], d3c, sem).wait()
            pltpu.sync_copy(a_s.at[pl.ds(ebase, CH_A1)], d3a)

            def fin(j, _):
                sl = pl.ds(16 * j, 16)
                d3a[sl] = d3a[sl] * (d3b[sl] * d3c[sl])
                return 0
            lax.fori_loop(0, CH_A1 // 16, fin, 0, unroll=4)
            pltpu.sync_copy(d3a, a_hbm.at[pl.ds(ebase, CH_A1)])
            return 0
        lax.fori_loop(0, EPT_A1 // CH_A1, phase3, 0)


@functools.partial(
    pl.kernel,
    out_type=(
        jax.ShapeDtypeStruct((NC, N_PAD, OUT), jnp.float32),
        jax.ShapeDtypeStruct((N_PAD, OUT), jnp.float32),
    ),
    mesh=_SC_MESH,
    compiler_params=pltpu.CompilerParams(use_tc_tiling_on_sc=False),
    scratch_types=[
        pltpu.VMEM_SHARED((N_PAD, OUT), jnp.float32),
        pltpu.VMEM_SHARED((N_PAD, OUT), jnp.float32),
        [pltpu.VMEM((CH_B,), jnp.int32)] * 3,
        [pltpu.VMEM((CH_B,), jnp.int32)] * 3,
        [pltpu.VMEM((CH_B,), jnp.float32)] * 3,
        [pltpu.VMEM((CH_B, OUT), jnp.float32)] * 2,
        pltpu.VMEM((ROWS_PER_TILE, OUT), jnp.float32),
        pltpu.VMEM((ROWS_PER_TILE, OUT), jnp.float32),
        pltpu.VMEM((16,), jnp.float32),
        [pltpu.SemaphoreType.DMA] * 3,
        [pltpu.SemaphoreType.DMA] * 2,
        [pltpu.SemaphoreType.DMA] * 2,
        pltpu.SemaphoreType.DMA,
    ],
)
def _spmm_kernel(parts_hbm, acc_hbm, ck_hbm, src_hbm, dst_hbm, a_hbm,
                 out_hbm, accout_hbm,
                 v_s, nv_s, idx_src, idx_dst, coeff, rows, p0buf, p1buf,
                 ckbuf, sem_l, sem_g, sem_s, sem0):
    cid = lax.axis_index("c")
    tid = lax.axis_index("s")
    nbase = tid * ROWS_PER_TILE
    half = ROWS_PER_TILE // 2

    d1 = pltpu.async_copy(parts_hbm.at[0, pl.ds(nbase, ROWS_PER_TILE)],
                          p0buf, sem0)
    d2 = pltpu.async_copy(parts_hbm.at[1, pl.ds(nbase, ROWS_PER_TILE)],
                          p1buf, sem0)
    d3 = pltpu.async_copy(ck_hbm, ckbuf, sem0)
    d1.wait(); d2.wait(); d3.wait()

    def comb(j, _):
        p0buf[j, :] = p0buf[j, :] + p1buf[j, :]
        return 0
    lax.fori_loop(0, ROWS_PER_TILE, comb, 0, unroll=8)
    pltpu.sync_copy(p0buf, v_s.at[pl.ds(nbase, ROWS_PER_TILE)])

    abase = nbase + cid * half
    aoff = cid * half
    pltpu.sync_copy(acc_hbm.at[pl.ds(abase, half)],
                    p1buf.at[pl.ds(0, half)])
    ck = ckbuf[pl.ds(0, 16)]

    def accu(j, _):
        p1buf[j, :] = p1buf[j, :] + ck * p0buf[aoff + j, :]
        return 0
    lax.fori_loop(0, half, accu, 0, unroll=8)
    pltpu.sync_copy(p1buf.at[pl.ds(0, half)],
                    accout_hbm.at[pl.ds(abase, half)])

    def z(j, _):
        p1buf[j, :] = jnp.zeros((16,), jnp.float32)
        return 0
    lax.fori_loop(0, ROWS_PER_TILE, z, 0, unroll=8)
    pltpu.sync_copy(p1buf, nv_s.at[pl.ds(nbase, ROWS_PER_TILE)])
    plsc.subcore_barrier()

    NCHUNK = EPT_B // CH_B
    tbase = (cid * NS + tid) * EPT_B

    def load_lin(i):
        q = i % 3
        ebase = tbase + i * CH_B
        return (pltpu.async_copy(src_hbm.at[pl.ds(ebase, CH_B)],
                                 idx_src[q], sem_l[q]),
                pltpu.async_copy(dst_hbm.at[pl.ds(ebase, CH_B)],
                                 idx_dst[q], sem_l[q]),
                pltpu.async_copy(a_hbm.at[pl.ds(ebase, CH_B)],
                                 coeff[q], sem_l[q]))

    def gather(i):
        p, q = i % 2, i % 3
        return pltpu.async_copy(v_s.at[idx_src[q]], rows[p], sem_g[p])

    def scatter(i):
        p, q = i % 2, i % 3
        return pltpu.async_copy(rows[p], nv_s.at[idx_dst[q]], sem_s[p],
                                add=True)

    def scale(i):
        p, q = i % 2, i % 3

        def body(g, _):
            coefs = coeff[q][pl.ds(16 * g, 16)]
            for lane in range(16):
                j = 16 * g + lane
                rows[p][j, :] = rows[p][j, :] * coefs[lane]
            return 0
        lax.fori_loop(0, CH_B // 16, body, 0)

    lin = [None] * NCHUNK
    gat = [None] * NCHUNK
    sca = [None] * NCHUNK
    lin[0] = load_lin(0)
    lin[1] = load_lin(1)
    for d in lin[0]:
        d.wait()
    gat[0] = gather(0)
    for i in range(NCHUNK):
        gat[i].wait()
        if i >= 1:
            sca[i - 1].wait()
        if i + 2 < NCHUNK:
            lin[i + 2] = load_lin(i + 2)
        if i + 1 < NCHUNK:
            for d in lin[i + 1]:
                d.wait()
            gat[i + 1] = gather(i + 1)
        scale(i)
        sca[i] = scatter(i)
    sca[NCHUNK - 1].wait()
    plsc.subcore_barrier()

    pltpu.sync_copy(nv_s.at[pl.ds(nbase, ROWS_PER_TILE)],
                    out_hbm.at[cid, pl.ds(nbase, ROWS_PER_TILE)])


def kernel(x, edge_index, perm, dist_w, W1, b1, W2, b2):
    x_pad = jnp.concatenate(
        [x, jnp.zeros((N_PAD - N, INP), jnp.float32)], axis=0)
    parts = _mlp(x_pad, W1, b1, W2, b2)

    pad_e = jnp.zeros((E_PAD - E,), jnp.int32)
    src_p = jnp.concatenate([edge_index[0].astype(jnp.int32), pad_e])
    dst_p = jnp.concatenate([edge_index[1].astype(jnp.int32), pad_e])
    dw = dist_w[:, 0]
    a_e = _prep_kernel(perm.astype(jnp.int32), dw, src_p, dst_p)

    acc = jnp.zeros((N_PAD, OUT), jnp.float32)
    for j in range(K_TAYLOR + 1):
        ckvec = jnp.asarray(
            np.full((16,), math.exp(-1.0) / math.factorial(j), np.float32))
        parts, acc = _spmm_kernel(parts, acc, ckvec, src_p, dst_p, a_e)

    return acc[:N].T


# speedup vs baseline: 576.5756x; 1.0800x over previous
import functools
import math

import jax
import jax.numpy as jnp
import numpy as np
from jax import lax
from jax.experimental import pallas as pl
from jax.experimental.pallas import tpu as pltpu
from jax.experimental.pallas import tpu_sc as plsc

N = 10000
E0 = 320000
INP = 128
OUT = 16
K_TAYLOR = 7

NC, NS = 2, 16
N_PAD = 10240
ROWS_PER_TILE = N_PAD // NS
E = E0 + N
E_PAD = 330240
EPT_B = E_PAD // (NC * NS)
CH_B = 2064
EPT_A0 = E0 // NS
CH_A0 = 2000
EPT_A1 = E_PAD // NS
CH_A1 = 2064


def _mlp_body(x_ref, w1t_ref, b1_ref, w2t_ref, b2_ref, o_ref):
    h = jnp.dot(x_ref[...], w1t_ref[...], preferred_element_type=jnp.float32)
    h = jnp.maximum(h + b1_ref[...], 0.0)
    o = jnp.dot(h, w2t_ref[...], preferred_element_type=jnp.float32)
    o_ref[0, :, :] = o + b2_ref[...]
    o_ref[1, :, :] = jnp.zeros((N_PAD, OUT), jnp.float32)


def _mlp(x_pad, W1, b1, W2, b2):
    return pl.pallas_call(
        _mlp_body,
        out_shape=jax.ShapeDtypeStruct((NC, N_PAD, OUT), jnp.float32),
    )(x_pad, W1.T, b1.reshape(1, -1), W2.T, b2.reshape(1, -1))


def _rsqrt_newton(x):
    i = lax.bitcast_convert_type(x, jnp.int32)
    i = jnp.int32(0x5F3759DF) - lax.shift_right_arithmetic(i, 1)
    y = lax.bitcast_convert_type(i, jnp.float32)
    for _ in range(4):
        y = y * (1.5 - 0.5 * x * y * y)
    return y


_SC_MESH = plsc.VectorSubcoreMesh(core_axis_name="c", subcore_axis_name="s")


@functools.partial(
    pl.kernel,
    out_type=jax.ShapeDtypeStruct((E_PAD,), jnp.float32),
    mesh=_SC_MESH,
    compiler_params=pltpu.CompilerParams(use_tc_tiling_on_sc=False),
    scratch_types=[
        pltpu.VMEM_SHARED((E_PAD,), jnp.float32),
        pltpu.VMEM_SHARED((E0,), jnp.float32),
        pltpu.VMEM_SHARED((N_PAD,), jnp.float32),
        pltpu.VMEM((CH_A0,), jnp.int32),
        pltpu.VMEM((CH_A0,), jnp.int32),
        pltpu.VMEM((CH_A0,), jnp.float32),
        pltpu.VMEM((CH_A0,), jnp.float32),
        pltpu.VMEM((CH_A1,), jnp.int32),
        pltpu.VMEM((CH_A1,), jnp.int32),
        pltpu.VMEM((CH_A1,), jnp.float32),
        pltpu.VMEM((CH_A1,), jnp.float32),
        pltpu.VMEM((CH_A1,), jnp.float32),
        pltpu.VMEM((ROWS_PER_TILE,), jnp.float32),
        pltpu.SemaphoreType.DMA,
    ],
)
def _prep_kernel(perm_hbm, dw_hbm, src_hbm, dst_hbm, a_hbm,
                 a_s, dw_s, deg_s, idx1, idxd1, d1a, d1b,
                 idx3a, idx3b, d3a, d3b, d3c, ndeg, sem):
    cid = lax.axis_index("c")
    tid = lax.axis_index("s")
    lanes = lax.iota(jnp.int32, 16)

    @pl.when(cid == 0)
    def _():
        nbase = tid * ROWS_PER_TILE
        pltpu.sync_copy(dw_hbm.at[pl.ds(tid * EPT_A0, EPT_A0)],
                        dw_s.at[pl.ds(tid * EPT_A0, EPT_A0)])
        for j in range(ROWS_PER_TILE // 16):
            ndeg[pl.ds(16 * j, 16)] = jnp.full((16,), 1.0, jnp.float32)
        pltpu.sync_copy(ndeg, deg_s.at[pl.ds(nbase, ROWS_PER_TILE)])
        for j in range(ROWS_PER_TILE // 16):
            gidx = nbase + 16 * j + lanes
            ndeg[pl.ds(16 * j, 16)] = jnp.where(
                gidx < N, 1.0, 0.0).astype(jnp.float32)
        pltpu.sync_copy(ndeg, a_s.at[pl.ds(E0 + nbase, ROWS_PER_TILE)])
        plsc.subcore_barrier()

        def phase1(i, _):
            ebase = tid * EPT_A0 + i * CH_A0
            pltpu.sync_copy(dw_hbm.at[pl.ds(ebase, CH_A0)], d1a)
            pltpu.sync_copy(perm_hbm.at[pl.ds(ebase, CH_A0)], idx1)
            pltpu.async_copy(dw_s.at[idx1], d1b, sem).wait()

            def sig(j, _):
                u = (d1a[pl.ds(16 * j, 16)] + d1b[pl.ds(16 * j, 16)]) * 0.5
                s = 1.0 / (1.0 + jnp.exp(-u))
                d1a[pl.ds(16 * j, 16)] = s
                return 0
            lax.fori_loop(0, CH_A0 // 16, sig, 0, unroll=4)
            pltpu.sync_copy(d1a, a_s.at[pl.ds(ebase, CH_A0)])
            pltpu.sync_copy(dst_hbm.at[pl.ds(ebase, CH_A0)], idxd1)
            pltpu.sync_copy(d1a, deg_s.at[idxd1], add=True)
            return 0
        lax.fori_loop(0, EPT_A0 // CH_A0, phase1, 0)
        plsc.subcore_barrier()

        nbase = tid * ROWS_PER_TILE
        pltpu.sync_copy(deg_s.at[pl.ds(nbase, ROWS_PER_TILE)], ndeg)

        def rsq(j, _):
            ndeg[pl.ds(16 * j, 16)] = _rsqrt_newton(ndeg[pl.ds(16 * j, 16)])
            return 0
        lax.fori_loop(0, ROWS_PER_TILE // 16, rsq, 0, unroll=4)
        pltpu.sync_copy(ndeg, deg_s.at[pl.ds(nbase, ROWS_PER_TILE)])
        plsc.subcore_barrier()

        def phase3(i, _):
            ebase = tid * EPT_A1 + i * CH_A1
            pltpu.sync_copy(src_hbm.at[pl.ds(ebase, CH_A1)], idx3a)
            pltpu.sync_copy(dst_hbm.at[pl.ds(ebase, CH_A1)], idx3b)
            pltpu.async_copy(deg_s.at[idx3a], d3b, sem).wait()
            pltpu.async_copy(deg_s.at[idx3b], d3c, sem).wait()
            pltpu.sync_copy(a_s.at[pl.ds(ebase, CH_A1)], d3a)

            def fin(j, _):
                sl = pl.ds(16 * j, 16)
                d3a[sl] = d3a[sl] * (d3b[sl] * d3c[sl])
                return 0
            lax.fori_loop(0, CH_A1 // 16, fin, 0, unroll=4)
            pltpu.sync_copy(d3a, a_hbm.at[pl.ds(ebase, CH_A1)])
            return 0
        lax.fori_loop(0, EPT_A1 // CH_A1, phase3, 0)


@functools.partial(
    pl.kernel,
    out_type=(
        jax.ShapeDtypeStruct((NC, N_PAD, OUT), jnp.float32),
        jax.ShapeDtypeStruct((N_PAD, OUT), jnp.float32),
    ),
    mesh=_SC_MESH,
    compiler_params=pltpu.CompilerParams(use_tc_tiling_on_sc=False),
    scratch_types=[
        pltpu.VMEM_SHARED((N_PAD, OUT), jnp.float32),
        pltpu.VMEM_SHARED((N_PAD, OUT), jnp.float32),
        [pltpu.VMEM((CH_B,), jnp.int32)] * 3,
        [pltpu.VMEM((CH_B,), jnp.int32)] * 3,
        [pltpu.VMEM((CH_B,), jnp.float32)] * 3,
        [pltpu.VMEM((CH_B, OUT), jnp.float32)] * 2,
        pltpu.VMEM((ROWS_PER_TILE, OUT), jnp.float32),
        pltpu.VMEM((ROWS_PER_TILE, OUT), jnp.float32),
        pltpu.VMEM((16,), jnp.float32),
        [pltpu.SemaphoreType.DMA] * 3,
        [pltpu.SemaphoreType.DMA] * 2,
        [pltpu.SemaphoreType.DMA] * 2,
        pltpu.SemaphoreType.DMA,
    ],
)
def _spmm_kernel(parts_hbm, acc_hbm, ck_hbm, src_hbm, dst_hbm, a_hbm,
                 out_hbm, accout_hbm,
                 v_s, nv_s, idx_src, idx_dst, coeff, rows, p0buf, p1buf,
                 ckbuf, sem_l, sem_g, sem_s, sem0):
    cid = lax.axis_index("c")
    tid = lax.axis_index("s")
    nbase = tid * ROWS_PER_TILE
    half = ROWS_PER_TILE // 2

    d1 = pltpu.async_copy(parts_hbm.at[0, pl.ds(nbase, ROWS_PER_TILE)],
                          p0buf, sem0)
    d2 = pltpu.async_copy(parts_hbm.at[1, pl.ds(nbase, ROWS_PER_TILE)],
                          p1buf, sem0)
    d3 = pltpu.async_copy(ck_hbm, ckbuf, sem0)
    d1.wait(); d2.wait(); d3.wait()

    def comb(j, _):
        p0buf[j, :] = p0buf[j, :] + p1buf[j, :]
        return 0
    lax.fori_loop(0, ROWS_PER_TILE, comb, 0, unroll=8)
    pltpu.sync_copy(p0buf, v_s.at[pl.ds(nbase, ROWS_PER_TILE)])

    abase = nbase + cid * half
    aoff = cid * half
    pltpu.sync_copy(acc_hbm.at[pl.ds(abase, half)],
                    p1buf.at[pl.ds(0, half)])
    ck = ckbuf[pl.ds(0, 16)]

    def accu(j, _):
        p1buf[j, :] = p1buf[j, :] + ck * p0buf[aoff + j, :]
        return 0
    lax.fori_loop(0, half, accu, 0, unroll=8)
    pltpu.sync_copy(p1buf.at[pl.ds(0, half)],
                    accout_hbm.at[pl.ds(abase, half)])

    def z(j, _):
        p1buf[j, :] = jnp.zeros((16,), jnp.float32)
        return 0
    lax.fori_loop(0, ROWS_PER_TILE, z, 0, unroll=8)
    pltpu.sync_copy(p1buf, nv_s.at[pl.ds(nbase, ROWS_PER_TILE)])
    plsc.subcore_barrier()

    NCHUNK = EPT_B // CH_B
    tbase = (cid * NS + tid) * EPT_B

    def load_lin(i):
        q = i % 3
        ebase = tbase + i * CH_B
        return (pltpu.async_copy(src_hbm.at[pl.ds(ebase, CH_B)],
                                 idx_src[q], sem_l[q]),
                pltpu.async_copy(dst_hbm.at[pl.ds(ebase, CH_B)],
                                 idx_dst[q], sem_l[q]),
                pltpu.async_copy(a_hbm.at[pl.ds(ebase, CH_B)],
                                 coeff[q], sem_l[q]))

    def gather(i):
        p, q = i % 2, i % 3
        return pltpu.async_copy(v_s.at[idx_src[q]], rows[p], sem_g[p])

    def scatter(i):
        p, q = i % 2, i % 3
        return pltpu.async_copy(rows[p], nv_s.at[idx_dst[q]], sem_s[p],
                                add=True)

    def scale(i):
        p, q = i % 2, i % 3

        def body(g, _):
            coefs = coeff[q][pl.ds(16 * g, 16)]
            for lane in range(16):
                j = 16 * g + lane
                rows[p][j, :] = rows[p][j, :] * coefs[lane]
            return 0
        lax.fori_loop(0, CH_B // 16, body, 0)

    lin = [None] * NCHUNK
    gat = [None] * NCHUNK
    sca = [None] * NCHUNK
    lin[0] = load_lin(0)
    lin[1] = load_lin(1)
    for d in lin[0]:
        d.wait()
    gat[0] = gather(0)
    for i in range(NCHUNK):
        gat[i].wait()
        if i >= 1:
            sca[i - 1].wait()
        if i + 2 < NCHUNK:
            lin[i + 2] = load_lin(i + 2)
        if i + 1 < NCHUNK:
            for d in lin[i + 1]:
                d.wait()
            gat[i + 1] = gather(i + 1)
        scale(i)
        sca[i] = scatter(i)
    sca[NCHUNK - 1].wait()
    plsc.subcore_barrier()

    pltpu.sync_copy(nv_s.at[pl.ds(nbase, ROWS_PER_TILE)],
                    out_hbm.at[cid, pl.ds(nbase, ROWS_PER_TILE)])


def kernel(x, edge_index, perm, dist_w, W1, b1, W2, b2):
    x_pad = jnp.concatenate(
        [x, jnp.zeros((N_PAD - N, INP), jnp.float32)], axis=0)
    parts = _mlp(x_pad, W1, b1, W2, b2)

    pad_e = jnp.zeros((E_PAD - E,), jnp.int32)
    src_p = jnp.concatenate([edge_index[0].astype(jnp.int32), pad_e])
    dst_p = jnp.concatenate([edge_index[1].astype(jnp.int32), pad_e])
    dw = dist_w[:, 0]
    a_e = _prep_kernel(perm.astype(jnp.int32), dw, src_p, dst_p)

    acc = jnp.zeros((N_PAD, OUT), jnp.float32)
    for j in range(K_TAYLOR + 1):
        ckvec = jnp.asarray(
            np.full((16,), math.exp(-1.0) / math.factorial(j), np.float32))
        parts, acc = _spmm_kernel(parts, acc, ckvec, src_p, dst_p, a_e)

    return acc[:N].T
